# Initial kernel scaffold; baseline (speedup 1.0000x reference)
import jax
import jax.numpy as jnp
from jax.experimental import pallas as pl


def kernel(V, F):
    raise NotImplementedError("write your pallas kernel here")


# speedup vs baseline: 68.0275x; 68.0275x over previous
import functools

import jax
import jax.numpy as jnp
from jax import lax
from jax.experimental import pallas as pl
from jax.experimental.pallas import tpu as pltpu
from jax.experimental.pallas import tpu_sc as plsc

_NC = 2
_NS = 16
_NW = _NC * _NS
_CHUNK = 128


def _rsqrt(x):
    y = plsc.bitcast(jnp.int32(0x5F3759DF) - (plsc.bitcast(x, jnp.int32) >> 1),
                     jnp.float32)
    xh = x * 0.5
    for _ in range(3):
        y = y * (1.5 - xh * y * y)
    return y


def _sc_body(cpw, x_hbm, f0_hbm, f1_hbm, f2_hbm, zero_hbm, out0_hbm, out1_hbm,
             acc, iv, gv, rv, ov, sem):
    c = lax.axis_index("c")
    s = lax.axis_index("s")

    @pl.when(s == 0)
    def _():
        pltpu.sync_copy(zero_hbm, acc)

    plsc.subcore_barrier()

    w = c * _NS + s

    def chunk_body(k, carry):
        base = (w * cpw + k) * _CHUNK
        base = pl.multiple_of(base, _CHUNK)
        pltpu.sync_copy(f0_hbm.at[pl.ds(base, _CHUNK)], iv[0])
        pltpu.sync_copy(f1_hbm.at[pl.ds(base, _CHUNK)], iv[1])
        pltpu.sync_copy(f2_hbm.at[pl.ds(base, _CHUNK)], iv[2])
        for j in range(_CHUNK // 16):
            sl = pl.ds(j * 16, 16)
            for v in range(3):
                i3 = iv[v][sl] * 3
                gv[3 * v + 0][sl] = i3
                gv[3 * v + 1][sl] = i3 + 1
                gv[3 * v + 2][sl] = i3 + 2
        cps = [pltpu.async_copy(x_hbm.at[gv[t]], rv[t], sem) for t in range(9)]
        for cp in cps:
            cp.wait()
        for j in range(_CHUNK // 16):
            sl = pl.ds(j * 16, 16)
            v1 = [rv[t][sl] for t in range(3)]
            v2 = [rv[3 + t][sl] for t in range(3)]
            v3 = [rv[6 + t][sl] for t in range(3)]
            d1 = [v2[t] - v3[t] for t in range(3)]
            d2 = [v3[t] - v1[t] for t in range(3)]
            d3 = [v1[t] - v2[t] for t in range(3)]
            q1 = d1[0] * d1[0] + d1[1] * d1[1] + d1[2] * d1[2]
            q2 = d2[0] * d2[0] + d2[1] * d2[1] + d2[2] * d2[2]
            q3 = d3[0] * d3[0] + d3[1] * d3[1] + d3[2] * d3[2]
            l1 = q1 * _rsqrt(q1)
            l2 = q2 * _rsqrt(q2)
            l3 = q3 * _rsqrt(q3)
            sp = (l1 + l2 + l3) * 0.5
            ins = sp * (sp - l1) * (sp - l2) * (sp - l3)
            ins = jnp.maximum(ins, 0.0)
            area2 = 2.0 * (ins * _rsqrt(ins))
            recip = 0.25 / (area2 + 1e-10)
            recip = jnp.where(area2 == 0.0, 0.0, recip)
            w0 = (q2 + q3 - q1) * recip
            w1 = (q1 + q3 - q2) * recip
            w2 = (q1 + q2 - q3) * recip
            for t in range(3):
                ov[t][sl] = w1 * d2[t] - w2 * d3[t]
                ov[3 + t][sl] = w2 * d3[t] - w0 * d1[t]
                ov[6 + t][sl] = w0 * d1[t] - w1 * d2[t]
        for t in range(9):
            pltpu.sync_copy(ov[t], acc.at[gv[t]], add=True)
        return carry

    lax.fori_loop(0, cpw, chunk_body, 0)
    plsc.subcore_barrier()

    @pl.when(s == 0)
    def _():
        @pl.when(c == 0)
        def _():
            pltpu.sync_copy(acc, out0_hbm)

        @pl.when(c == 1)
        def _():
            pltpu.sync_copy(acc, out1_hbm)


def _combine_body(a_ref, b_ref, o_ref):
    o_ref[...] = a_ref[...] + b_ref[...]


@jax.jit
def kernel(V, F):
    B, N, _ = V.shape
    Fn = F.shape[1]
    BN = B * N
    T = B * Fn
    cpw = -(-T // (_NW * _CHUNK))
    TP = _NW * cpw * _CHUNK

    x = V.reshape(BN * 3)
    offs = (jnp.arange(B, dtype=F.dtype) * jnp.asarray(N, F.dtype))[:, None, None]
    bf = (F + offs).reshape(T, 3)
    pad = TP - T
    f0 = jnp.concatenate([bf[:, 0], jnp.zeros((pad,), bf.dtype)])
    f1 = jnp.concatenate([bf[:, 1], jnp.zeros((pad,), bf.dtype)])
    f2 = jnp.concatenate([bf[:, 2], jnp.zeros((pad,), bf.dtype)])
    zero = jnp.zeros((BN * 3,), jnp.float32)

    mesh = plsc.VectorSubcoreMesh(core_axis_name="c", subcore_axis_name="s",
                                  num_cores=_NC, num_subcores=_NS)
    sc_call = pl.kernel(
        functools.partial(_sc_body, cpw),
        out_type=(jax.ShapeDtypeStruct((BN * 3,), jnp.float32),
                  jax.ShapeDtypeStruct((BN * 3,), jnp.float32)),
        mesh=mesh,
        scratch_types=[
            pltpu.VMEM_SHARED((BN * 3,), jnp.float32),
            [pltpu.VMEM((_CHUNK,), jnp.int32) for _ in range(3)],
            [pltpu.VMEM((_CHUNK,), jnp.int32) for _ in range(9)],
            [pltpu.VMEM((_CHUNK,), jnp.float32) for _ in range(9)],
            [pltpu.VMEM((_CHUNK,), jnp.float32) for _ in range(9)],
            pltpu.SemaphoreType.DMA,
        ],
        compiler_params=pltpu.CompilerParams(needs_layout_passes=False),
    )
    p0, p1 = sc_call(x, f0, f1, f2, zero)

    L = BN * 3
    Lp = -(-L // 512) * 512
    q0 = jnp.pad(p0, (0, Lp - L)).reshape(-1, 512)
    q1 = jnp.pad(p1, (0, Lp - L)).reshape(-1, 512)
    out = pl.pallas_call(
        _combine_body,
        out_shape=jax.ShapeDtypeStruct(q0.shape, jnp.float32),
    )(q0, q1)
    return out.reshape(-1)[:L].reshape(B, N, 3)
